# baseline (speedup 1.0000x reference)
import functools

import jax
import jax.numpy as jnp
from jax import lax
from jax.experimental import pallas as pl
from jax.experimental.pallas import tpu as pltpu
from jax.experimental.pallas import tpu_sc as plsc

N = 10000
E = 320000
D = 128
NP = 10240
NW = 32
EPW = E // NW
CW = 256
CH = (EPW + CW - 1) // CW
EPP = CH * CW
DA = 144
NACC = 10160
ACC_PER_TILE = NACC // 16
PAD_DST = 10100
ROWS_PER_TILE = NP // 16


def _mm_body(x_ref, ws_ref, wl_ref, bl_ref, wd_ref, attd_ref, atts_ref,
             h_ref, skip_ref, as_ref, ad_ref, mas_ref, mad_ref):
    i = pl.program_id(0)
    xb = x_ref[...]
    h = jnp.dot(xb, ws_ref[...], preferred_element_type=jnp.float32)
    h_ref[:, :D] = h
    lane = jax.lax.broadcasted_iota(jnp.int32, (xb.shape[0], DA - D), 1)
    h_ref[:, D:] = jnp.where(lane == 0, 1.0, 0.0)
    skip_ref[...] = (jnp.dot(xb, wl_ref[...], preferred_element_type=jnp.float32)
                     + bl_ref[...][None, :])
    a_s = jnp.sum(h * atts_ref[...][None, :], axis=1)
    as_ref[...] = a_s
    wdv = jnp.sum(wd_ref[...] * attd_ref[...][None, :], axis=1)
    a_d = jnp.sum(xb * wdv[None, :], axis=1)
    ad_ref[...] = a_d

    @pl.when(i == 0)
    def _():
        mas_ref[0, 0] = -jnp.inf
        mad_ref[0, 0] = -jnp.inf

    mas_ref[0, 0] = jnp.maximum(mas_ref[0, 0], jnp.max(a_s))
    mad_ref[0, 0] = jnp.maximum(mad_ref[0, 0], jnp.max(a_d))


def _mm(x, w_src, wl, bl, w_dst, att_dst, att_src):
    blk = 512
    grid = NP // blk
    return pl.pallas_call(
        _mm_body,
        grid=(grid,),
        in_specs=[
            pl.BlockSpec((blk, D), lambda i: (i, 0)),
            pl.BlockSpec((D, D), lambda i: (0, 0)),
            pl.BlockSpec((D, D), lambda i: (0, 0)),
            pl.BlockSpec((D,), lambda i: (0,)),
            pl.BlockSpec((D, D), lambda i: (0, 0)),
            pl.BlockSpec((D,), lambda i: (0,)),
            pl.BlockSpec((D,), lambda i: (0,)),
        ],
        out_specs=[
            pl.BlockSpec((blk, DA), lambda i: (i, 0)),
            pl.BlockSpec((blk, D), lambda i: (i, 0)),
            pl.BlockSpec((blk,), lambda i: (i,)),
            pl.BlockSpec((blk,), lambda i: (i,)),
            pl.BlockSpec((1, 1), lambda i: (0, 0), memory_space=pltpu.SMEM),
            pl.BlockSpec((1, 1), lambda i: (0, 0), memory_space=pltpu.SMEM),
        ],
        out_shape=[
            jax.ShapeDtypeStruct((NP, DA), jnp.float32),
            jax.ShapeDtypeStruct((NP, D), jnp.float32),
            jax.ShapeDtypeStruct((NP,), jnp.float32),
            jax.ShapeDtypeStruct((NP,), jnp.float32),
            jax.ShapeDtypeStruct((1, 1), jnp.float32),
            jax.ShapeDtypeStruct((1, 1), jnp.float32),
        ],
    )(x, w_src, wl, bl, w_dst, att_dst, att_src)


NI = 3


def _sc_edge_body(h_hbm, as_hbm, ad_hbm, idx_hbm, c_hbm, zr_hbm,
                  acc_out,
                  acc_sh, idxv, asg, adg, pbuf, rows, cv, gsem, asem, ssem):
    ci = lax.axis_index("c")
    si = lax.axis_index("s")
    wid = ci * 16 + si

    pltpu.sync_copy(zr_hbm, acc_sh.at[pl.ds(si * ACC_PER_TILE, ACC_PER_TILE)])
    pltpu.sync_copy(c_hbm, cv)
    plsc.subcore_barrier()

    cvec = cv[...]
    HB = CW // 128

    def chunk_body(i, carry):
        pltpu.sync_copy(idx_hbm.at[wid, i], idxv)
        rcps, acps = [], []
        for hf in range(HB):
            rcps.append(pltpu.async_copy(
                h_hbm.at[idxv.at[0, hf]],
                rows.at[pl.ds(hf * 128, 128)], gsem))
            acps.append(pltpu.async_copy(
                as_hbm.at[idxv.at[0, hf]],
                asg.at[pl.ds(hf * 128, 128)], asem))
            acps.append(pltpu.async_copy(
                ad_hbm.at[idxv.at[1, hf]],
                adg.at[pl.ds(hf * 128, 128)], asem))
        for acp in acps:
            acp.wait()

        def vreg_body(k, c2):
            e = asg[pl.ds(k * 16, 16)] + adg[pl.ds(k * 16, 16)]
            e = jnp.where(e > 0, e, 0.2 * e)
            pbuf[pl.ds(k * 16, 16)] = jnp.exp(e - cvec)
            return c2

        lax.fori_loop(0, CW // 16, vreg_body, 0, unroll=True)
        for rcp in rcps:
            rcp.wait()

        @plsc.parallel_loop(0, CW, step=16)
        def row_body(g):
            pv = pbuf[pl.ds(g, 16)]
            for l in range(16):
                ps = pv[l]
                for k2 in range(DA // 16):
                    rows[g + l, pl.ds(k2 * 16, 16)] = (
                        rows[g + l, pl.ds(k2 * 16, 16)] * ps)

        scps = [pltpu.async_copy(rows.at[pl.ds(hf * 128, 128)],
                                 acc_sh.at[idxv.at[1, hf]], ssem, add=True)
                for hf in range(HB)]
        for scp in scps:
            scp.wait()
        return carry

    lax.fori_loop(0, CH, chunk_body, 0)
    plsc.subcore_barrier()

    sl = pl.ds(si * ACC_PER_TILE, ACC_PER_TILE)
    pltpu.sync_copy(acc_sh.at[sl], acc_out.at[ci, sl])


@functools.partial(
    pl.kernel,
    out_type=jax.ShapeDtypeStruct((2, NACC, DA), jnp.float32),
    mesh=plsc.VectorSubcoreMesh(core_axis_name="c", subcore_axis_name="s"),
    compiler_params=pltpu.CompilerParams(needs_layout_passes=False,
                                         use_tc_tiling_on_sc=False),
    scratch_types=[
        pltpu.VMEM_SHARED((NACC, DA), jnp.float32),
        pltpu.VMEM((2, CW // 128, 128), jnp.int32),
        pltpu.VMEM((CW,), jnp.float32),
        pltpu.VMEM((CW,), jnp.float32),
        pltpu.VMEM((CW,), jnp.float32),
        pltpu.VMEM((CW, DA), jnp.float32),
        pltpu.VMEM((16,), jnp.float32),
        pltpu.SemaphoreType.DMA,
        pltpu.SemaphoreType.DMA,
        pltpu.SemaphoreType.DMA,
    ],
)
def _sc_edge(*refs):
    _sc_edge_body(*refs)


def _comb_body(acc_ref, skip_ref, b_ref, out_ref):
    full = acc_ref[0, :, :] + acc_ref[1, :, :]
    num = full[:, :D]
    dn = full[:, D:D + 1] + 1e-16
    h = num / dn + skip_ref[...] + b_ref[...][None, :]
    out_ref[...] = jnp.maximum(h, 0.0)


def _comb(acc, skip, b):
    blk = 1016
    return pl.pallas_call(
        _comb_body,
        grid=(NACC // blk,),
        in_specs=[
            pl.BlockSpec((2, blk, DA), lambda i: (0, i, 0)),
            pl.BlockSpec((blk, D), lambda i: (i, 0)),
            pl.BlockSpec((D,), lambda i: (0,)),
        ],
        out_specs=pl.BlockSpec((blk, D), lambda i: (i, 0)),
        out_shape=jax.ShapeDtypeStruct((NACC, D), jnp.float32),
    )(acc, skip, b)


def _layer(x_pad, idx3, zr, w_src, w_dst, att_src, att_dst, b, wl, bl):
    h, skip, a_s, a_d, mas, mad = _mm(x_pad, w_src, wl, bl, w_dst, att_dst, att_src)
    cb = mas[0, 0] + mad[0, 0]
    c = jnp.where(cb > 0, cb, 0.2 * cb)
    cvec = jnp.full((16,), c, jnp.float32)
    acc = _sc_edge(h, a_s, a_d, idx3, cvec, zr)
    out = _comb(acc, skip[:NACC], b)
    return jnp.pad(out, ((0, NP - NACC), (0, 0)))


def kernel(x, edge_index, W1_src, W1_dst, att1_src, att1_dst, b1, Wl1, bl1,
           W2_src, W2_dst, att2_src, att2_dst, b2, Wl2, bl2):
    x_pad = jnp.pad(x, ((0, NP - N), (0, 0)))
    src = edge_index[0].astype(jnp.int32).reshape(NW, EPW)
    dst = edge_index[1].astype(jnp.int32).reshape(NW, EPW)
    pad = ((0, 0), (0, EPP - EPW))
    src3 = jnp.pad(src, pad, constant_values=NP - 1).reshape(NW, CH, CW // 128, 128)
    dst3 = jnp.pad(dst, pad, constant_values=PAD_DST).reshape(NW, CH, CW // 128, 128)
    idx3 = jnp.stack([src3, dst3], axis=2)
    zr = jnp.zeros((ACC_PER_TILE, DA), jnp.float32)

    h = _layer(x_pad, idx3, zr,
               W1_src, W1_dst, att1_src, att1_dst, b1, Wl1, bl1)
    out = _layer(h, idx3, zr,
                 W2_src, W2_dst, att2_src, att2_dst, b2, Wl2, bl2)
    return out[:N]


# speedup vs baseline: 1.3243x; 1.3243x over previous
import functools

import jax
import jax.numpy as jnp
from jax import lax
from jax.experimental import pallas as pl
from jax.experimental.pallas import tpu as pltpu
from jax.experimental.pallas import tpu_sc as plsc

N = 10000
E = 320000
D = 128
NP = 10240
NW = 32
EPW = E // NW
CW = 128
CH = (EPW + CW - 1) // CW
EPP = CH * CW
DA = 144
NACC = 10160
ACC_PER_TILE = NACC // 16
PAD_DST = 10100
ROWS_PER_TILE = NP // 16


def _mm_body(x_ref, ws_ref, wl_ref, bl_ref, wd_ref, attd_ref, atts_ref,
             h_ref, skip_ref, as_ref, ad_ref, mas_ref, mad_ref):
    i = pl.program_id(0)
    xb = x_ref[...]
    h = jnp.dot(xb, ws_ref[...], preferred_element_type=jnp.float32)
    h_ref[:, :D] = h
    lane = jax.lax.broadcasted_iota(jnp.int32, (xb.shape[0], DA - D), 1)
    h_ref[:, D:] = jnp.where(lane == 0, 1.0, 0.0)
    skip_ref[...] = (jnp.dot(xb, wl_ref[...], preferred_element_type=jnp.float32)
                     + bl_ref[...][None, :])
    a_s = jnp.sum(h * atts_ref[...][None, :], axis=1)
    as_ref[...] = a_s
    wdv = jnp.sum(wd_ref[...] * attd_ref[...][None, :], axis=1)
    a_d = jnp.sum(xb * wdv[None, :], axis=1)
    ad_ref[...] = a_d

    @pl.when(i == 0)
    def _():
        mas_ref[0, 0] = -jnp.inf
        mad_ref[0, 0] = -jnp.inf

    mas_ref[0, 0] = jnp.maximum(mas_ref[0, 0], jnp.max(a_s))
    mad_ref[0, 0] = jnp.maximum(mad_ref[0, 0], jnp.max(a_d))


def _mm(x, w_src, wl, bl, w_dst, att_dst, att_src):
    blk = 512
    grid = NP // blk
    return pl.pallas_call(
        _mm_body,
        grid=(grid,),
        in_specs=[
            pl.BlockSpec((blk, D), lambda i: (i, 0)),
            pl.BlockSpec((D, D), lambda i: (0, 0)),
            pl.BlockSpec((D, D), lambda i: (0, 0)),
            pl.BlockSpec((D,), lambda i: (0,)),
            pl.BlockSpec((D, D), lambda i: (0, 0)),
            pl.BlockSpec((D,), lambda i: (0,)),
            pl.BlockSpec((D,), lambda i: (0,)),
        ],
        out_specs=[
            pl.BlockSpec((blk, DA), lambda i: (i, 0)),
            pl.BlockSpec((blk, D), lambda i: (i, 0)),
            pl.BlockSpec((blk,), lambda i: (i,)),
            pl.BlockSpec((blk,), lambda i: (i,)),
            pl.BlockSpec((1, 1), lambda i: (0, 0), memory_space=pltpu.SMEM),
            pl.BlockSpec((1, 1), lambda i: (0, 0), memory_space=pltpu.SMEM),
        ],
        out_shape=[
            jax.ShapeDtypeStruct((NP, DA), jnp.float32),
            jax.ShapeDtypeStruct((NP, D), jnp.float32),
            jax.ShapeDtypeStruct((NP,), jnp.float32),
            jax.ShapeDtypeStruct((NP,), jnp.float32),
            jax.ShapeDtypeStruct((1, 1), jnp.float32),
            jax.ShapeDtypeStruct((1, 1), jnp.float32),
        ],
    )(x, w_src, wl, bl, w_dst, att_dst, att_src)


NI = 3


def _sc_edge_body(h_hbm, as_hbm, ad_hbm, idx_hbm, c_hbm, zr_hbm,
                  acc_out,
                  acc_sh, asv, adv, idxv, pbuf, rows, cv, gsem, isem):
    ci = lax.axis_index("c")
    si = lax.axis_index("s")
    wid = ci * 16 + si

    pltpu.sync_copy(zr_hbm, acc_sh.at[pl.ds(si * ACC_PER_TILE, ACC_PER_TILE)])
    pltpu.sync_copy(as_hbm, asv)
    pltpu.sync_copy(ad_hbm, adv)
    pltpu.sync_copy(c_hbm, cv)
    plsc.subcore_barrier()

    cvec = cv[...]

    pltpu.async_copy(idx_hbm.at[wid, 0], idxv.at[0], isem.at[0])

    def chunk_body(i, carry):
        par = lax.rem(i, 2)
        nxt = lax.rem(i + 1, 2)

        pltpu.make_async_copy(idx_hbm.at[wid, i], idxv.at[par],
                              isem.at[par]).wait()
        rcp = pltpu.async_copy(h_hbm.at[idxv.at[par, 0]], rows, gsem)

        @pl.when(i < CH - 1)
        def _():
            pltpu.async_copy(idx_hbm.at[wid, i + 1], idxv.at[nxt],
                             isem.at[nxt])

        def vreg_body(k, c2):
            sidx = idxv[par, 0, pl.ds(k * 16, 16)]
            didx = idxv[par, 1, pl.ds(k * 16, 16)]
            e = plsc.load_gather(asv, [sidx]) + plsc.load_gather(adv, [didx])
            e = jnp.where(e > 0, e, 0.2 * e)
            pbuf[pl.ds(k * 16, 16)] = jnp.exp(e - cvec)
            return c2

        lax.fori_loop(0, CW // 16, vreg_body, 0, unroll=True)
        rcp.wait()

        @plsc.parallel_loop(0, CW, step=16)
        def row_body(g):
            pv = pbuf[pl.ds(g, 16)]
            for l in range(16):
                ps = pv[l]
                for k2 in range(DA // 16):
                    rows[g + l, pl.ds(k2 * 16, 16)] = (
                        rows[g + l, pl.ds(k2 * 16, 16)] * ps)

        pltpu.sync_copy(rows, acc_sh.at[idxv.at[par, 1]], add=True)
        return carry

    lax.fori_loop(0, CH, chunk_body, 0)
    plsc.subcore_barrier()

    sl = pl.ds(si * ACC_PER_TILE, ACC_PER_TILE)
    pltpu.sync_copy(acc_sh.at[sl], acc_out.at[ci, sl])


@functools.partial(
    pl.kernel,
    out_type=jax.ShapeDtypeStruct((2, NACC, DA), jnp.float32),
    mesh=plsc.VectorSubcoreMesh(core_axis_name="c", subcore_axis_name="s"),
    compiler_params=pltpu.CompilerParams(needs_layout_passes=False,
                                         use_tc_tiling_on_sc=False),
    scratch_types=[
        pltpu.VMEM_SHARED((NACC, DA), jnp.float32),
        pltpu.VMEM((NP,), jnp.float32),
        pltpu.VMEM((NP,), jnp.float32),
        pltpu.VMEM((2, 2, CW), jnp.int32),
        pltpu.VMEM((CW,), jnp.float32),
        pltpu.VMEM((CW, DA), jnp.float32),
        pltpu.VMEM((16,), jnp.float32),
        pltpu.SemaphoreType.DMA,
        pltpu.SemaphoreType.DMA((2,)),
    ],
)
def _sc_edge(*refs):
    _sc_edge_body(*refs)


def _comb_body(acc_ref, skip_ref, b_ref, out_ref):
    full = acc_ref[0, :, :] + acc_ref[1, :, :]
    num = full[:, :D]
    dn = full[:, D:D + 1] + 1e-16
    h = num / dn + skip_ref[...] + b_ref[...][None, :]
    out_ref[...] = jnp.maximum(h, 0.0)


def _comb(acc, skip, b):
    blk = 1016
    return pl.pallas_call(
        _comb_body,
        grid=(NACC // blk,),
        in_specs=[
            pl.BlockSpec((2, blk, DA), lambda i: (0, i, 0)),
            pl.BlockSpec((blk, D), lambda i: (i, 0)),
            pl.BlockSpec((D,), lambda i: (0,)),
        ],
        out_specs=pl.BlockSpec((blk, D), lambda i: (i, 0)),
        out_shape=jax.ShapeDtypeStruct((NACC, D), jnp.float32),
    )(acc, skip, b)


def _layer(x_pad, idx3, zr, w_src, w_dst, att_src, att_dst, b, wl, bl):
    h, skip, a_s, a_d, mas, mad = _mm(x_pad, w_src, wl, bl, w_dst, att_dst, att_src)
    cb = mas[0, 0] + mad[0, 0]
    c = jnp.where(cb > 0, cb, 0.2 * cb)
    cvec = jnp.full((16,), c, jnp.float32)
    acc = _sc_edge(h, a_s, a_d, idx3, cvec, zr)
    out = _comb(acc, skip[:NACC], b)
    return jnp.pad(out, ((0, NP - NACC), (0, 0)))


def kernel(x, edge_index, W1_src, W1_dst, att1_src, att1_dst, b1, Wl1, bl1,
           W2_src, W2_dst, att2_src, att2_dst, b2, Wl2, bl2):
    x_pad = jnp.pad(x, ((0, NP - N), (0, 0)))
    src = edge_index[0].astype(jnp.int32).reshape(NW, EPW)
    dst = edge_index[1].astype(jnp.int32).reshape(NW, EPW)
    pad = ((0, 0), (0, EPP - EPW))
    src3 = jnp.pad(src, pad, constant_values=NP - 1).reshape(NW, CH, CW)
    dst3 = jnp.pad(dst, pad, constant_values=PAD_DST).reshape(NW, CH, CW)
    idx3 = jnp.stack([src3, dst3], axis=2)
    zr = jnp.zeros((ACC_PER_TILE, DA), jnp.float32)

    h = _layer(x_pad, idx3, zr,
               W1_src, W1_dst, att1_src, att1_dst, b1, Wl1, bl1)
    out = _layer(h, idx3, zr,
                 W2_src, W2_dst, att2_src, att2_dst, b2, Wl2, bl2)
    return out[:N]


# speedup vs baseline: 1.3301x; 1.0044x over previous
import functools

import jax
import jax.numpy as jnp
from jax import lax
from jax.experimental import pallas as pl
from jax.experimental.pallas import tpu as pltpu
from jax.experimental.pallas import tpu_sc as plsc

N = 10000
E = 320000
D = 128
NP = 10240
NW = 32
EPW = E // NW
CW = 128
CH = (EPW + CW - 1) // CW
EPP = CH * CW
DA = 144
NACC = 10112
ACC_PER_TILE = NACC // 16
PAD_DST = 10100
ROWS_PER_TILE = NP // 16


def _mm_body(x_ref, ws_ref, wl_ref, bl_ref, wd_ref, attd_ref, atts_ref,
             h_ref, skip_ref, as_ref, ad_ref, mas_ref, mad_ref):
    i = pl.program_id(0)
    xb = x_ref[...]
    h = jnp.dot(xb, ws_ref[...], preferred_element_type=jnp.float32)
    h_ref[:, :D] = h
    lane = jax.lax.broadcasted_iota(jnp.int32, (xb.shape[0], DA - D), 1)
    h_ref[:, D:] = jnp.where(lane == 0, 1.0, 0.0)
    skip_ref[...] = (jnp.dot(xb, wl_ref[...], preferred_element_type=jnp.float32)
                     + bl_ref[...][None, :])
    a_s = jnp.sum(h * atts_ref[...][None, :], axis=1)
    as_ref[...] = a_s
    wdv = jnp.sum(wd_ref[...] * attd_ref[...][None, :], axis=1)
    a_d = jnp.sum(xb * wdv[None, :], axis=1)
    ad_ref[...] = a_d

    @pl.when(i == 0)
    def _():
        mas_ref[0, 0] = -jnp.inf
        mad_ref[0, 0] = -jnp.inf

    mas_ref[0, 0] = jnp.maximum(mas_ref[0, 0], jnp.max(a_s))
    mad_ref[0, 0] = jnp.maximum(mad_ref[0, 0], jnp.max(a_d))


def _mm(x, w_src, wl, bl, w_dst, att_dst, att_src):
    blk = 512
    grid = NP // blk
    return pl.pallas_call(
        _mm_body,
        grid=(grid,),
        in_specs=[
            pl.BlockSpec((blk, D), lambda i: (i, 0)),
            pl.BlockSpec((D, D), lambda i: (0, 0)),
            pl.BlockSpec((D, D), lambda i: (0, 0)),
            pl.BlockSpec((D,), lambda i: (0,)),
            pl.BlockSpec((D, D), lambda i: (0, 0)),
            pl.BlockSpec((D,), lambda i: (0,)),
            pl.BlockSpec((D,), lambda i: (0,)),
        ],
        out_specs=[
            pl.BlockSpec((blk, DA), lambda i: (i, 0)),
            pl.BlockSpec((blk, D), lambda i: (i, 0)),
            pl.BlockSpec((blk,), lambda i: (i,)),
            pl.BlockSpec((blk,), lambda i: (i,)),
            pl.BlockSpec((1, 1), lambda i: (0, 0), memory_space=pltpu.SMEM),
            pl.BlockSpec((1, 1), lambda i: (0, 0), memory_space=pltpu.SMEM),
        ],
        out_shape=[
            jax.ShapeDtypeStruct((NP, DA), jnp.float32),
            jax.ShapeDtypeStruct((NP, D), jnp.float32),
            jax.ShapeDtypeStruct((NP,), jnp.float32),
            jax.ShapeDtypeStruct((NP,), jnp.float32),
            jax.ShapeDtypeStruct((1, 1), jnp.float32),
            jax.ShapeDtypeStruct((1, 1), jnp.float32),
        ],
    )(x, w_src, wl, bl, w_dst, att_dst, att_src)


NI = 3


def _sc_edge_body(h_hbm, as_hbm, ad_hbm, idx_hbm, c_hbm, zr_hbm,
                  acc_out,
                  acc_sh, asv, adv, idxv, pbuf, rows, cv, gsem, isem):
    ci = lax.axis_index("c")
    si = lax.axis_index("s")
    wid = ci * 16 + si

    pltpu.sync_copy(zr_hbm, acc_sh.at[pl.ds(si * ACC_PER_TILE, ACC_PER_TILE)])
    pltpu.sync_copy(as_hbm, asv)
    pltpu.sync_copy(ad_hbm, adv)
    pltpu.sync_copy(c_hbm, cv)
    plsc.subcore_barrier()

    cvec = cv[...]

    def compute_p(slot, pb):
        def vreg_body(k, c2):
            sidx = idxv[slot, 0, pl.ds(k * 16, 16)]
            didx = idxv[slot, 1, pl.ds(k * 16, 16)]
            e = plsc.load_gather(asv, [sidx]) + plsc.load_gather(adv, [didx])
            e = jnp.where(e > 0, e, 0.2 * e)
            pbuf[pb, pl.ds(k * 16, 16)] = jnp.exp(e - cvec)
            return c2

        lax.fori_loop(0, CW // 16, vreg_body, 0, unroll=True)

    def scale_rows(pb):
        @plsc.parallel_loop(0, CW, step=16)
        def row_body(g):
            pv = pbuf[pb, pl.ds(g, 16)]
            for l in range(16):
                ps = pv[l]
                for k2 in range(DA // 16):
                    rows[g + l, pl.ds(k2 * 16, 16)] = (
                        rows[g + l, pl.ds(k2 * 16, 16)] * ps)

    pltpu.async_copy(idx_hbm.at[wid, 0], idxv.at[0], isem.at[0])
    pltpu.make_async_copy(idx_hbm.at[wid, 0], idxv.at[0], isem.at[0]).wait()
    rcp0 = pltpu.async_copy(h_hbm.at[idxv.at[0, 0]], rows, gsem)
    pltpu.async_copy(idx_hbm.at[wid, 1], idxv.at[1], isem.at[1])
    compute_p(0, 0)
    rcp0.wait()
    scale_rows(0)

    def chunk_body(i, carry):
        par = lax.rem(i, 2)
        nxt = lax.rem(i + 1, 2)
        slot = lax.rem(i, NI)
        nslot = lax.rem(i + 1, NI)
        fslot = lax.rem(i + 2, NI)

        scp = pltpu.async_copy(rows, acc_sh.at[idxv.at[slot, 1]], gsem,
                               add=True)

        @pl.when(i < CH - 1)
        def _():
            pltpu.make_async_copy(idx_hbm.at[wid, i + 1], idxv.at[nslot],
                                  isem.at[nslot]).wait()
            compute_p(nslot, nxt)

        scp.wait()

        @pl.when(i < CH - 1)
        def _():
            rcp = pltpu.async_copy(h_hbm.at[idxv.at[nslot, 0]], rows, gsem)

            @pl.when(i < CH - 2)
            def _():
                pltpu.async_copy(idx_hbm.at[wid, i + 2], idxv.at[fslot],
                                 isem.at[fslot])

            rcp.wait()
            scale_rows(nxt)

        return carry

    lax.fori_loop(0, CH, chunk_body, 0)
    plsc.subcore_barrier()

    sl = pl.ds(si * ACC_PER_TILE, ACC_PER_TILE)
    pltpu.sync_copy(acc_sh.at[sl], acc_out.at[ci, sl])


@functools.partial(
    pl.kernel,
    out_type=jax.ShapeDtypeStruct((2, NACC, DA), jnp.float32),
    mesh=plsc.VectorSubcoreMesh(core_axis_name="c", subcore_axis_name="s"),
    compiler_params=pltpu.CompilerParams(needs_layout_passes=False,
                                         use_tc_tiling_on_sc=False),
    scratch_types=[
        pltpu.VMEM_SHARED((NACC, DA), jnp.float32),
        pltpu.VMEM((NP,), jnp.float32),
        pltpu.VMEM((NP,), jnp.float32),
        pltpu.VMEM((NI, 2, CW), jnp.int32),
        pltpu.VMEM((2, CW), jnp.float32),
        pltpu.VMEM((CW, DA), jnp.float32),
        pltpu.VMEM((16,), jnp.float32),
        pltpu.SemaphoreType.DMA,
        pltpu.SemaphoreType.DMA((NI,)),
    ],
)
def _sc_edge(*refs):
    _sc_edge_body(*refs)


def _comb_body(acc_ref, skip_ref, b_ref, out_ref):
    full = acc_ref[0, :, :] + acc_ref[1, :, :]
    num = full[:, :D]
    dn = full[:, D:D + 1] + 1e-16
    h = num / dn + skip_ref[...] + b_ref[...][None, :]
    out_ref[...] = jnp.maximum(h, 0.0)


def _comb(acc, skip, b):
    blk = 1264
    return pl.pallas_call(
        _comb_body,
        grid=(NACC // blk,),
        in_specs=[
            pl.BlockSpec((2, blk, DA), lambda i: (0, i, 0)),
            pl.BlockSpec((blk, D), lambda i: (i, 0)),
            pl.BlockSpec((D,), lambda i: (0,)),
        ],
        out_specs=pl.BlockSpec((blk, D), lambda i: (i, 0)),
        out_shape=jax.ShapeDtypeStruct((NACC, D), jnp.float32),
    )(acc, skip, b)


def _layer(x_pad, idx3, zr, w_src, w_dst, att_src, att_dst, b, wl, bl):
    h, skip, a_s, a_d, mas, mad = _mm(x_pad, w_src, wl, bl, w_dst, att_dst, att_src)
    cb = mas[0, 0] + mad[0, 0]
    c = jnp.where(cb > 0, cb, 0.2 * cb)
    cvec = jnp.full((16,), c, jnp.float32)
    acc = _sc_edge(h, a_s, a_d, idx3, cvec, zr)
    out = _comb(acc, skip[:NACC], b)
    return jnp.pad(out, ((0, NP - NACC), (0, 0)))


def kernel(x, edge_index, W1_src, W1_dst, att1_src, att1_dst, b1, Wl1, bl1,
           W2_src, W2_dst, att2_src, att2_dst, b2, Wl2, bl2):
    x_pad = jnp.pad(x, ((0, NP - N), (0, 0)))
    src = edge_index[0].astype(jnp.int32).reshape(NW, EPW)
    dst = edge_index[1].astype(jnp.int32).reshape(NW, EPW)
    pad = ((0, 0), (0, EPP - EPW))
    src3 = jnp.pad(src, pad, constant_values=NP - 1).reshape(NW, CH, CW)
    dst3 = jnp.pad(dst, pad, constant_values=PAD_DST).reshape(NW, CH, CW)
    idx3 = jnp.stack([src3, dst3], axis=2)
    zr = jnp.zeros((ACC_PER_TILE, DA), jnp.float32)

    h = _layer(x_pad, idx3, zr,
               W1_src, W1_dst, att1_src, att1_dst, b1, Wl1, bl1)
    out = _layer(h, idx3, zr,
                 W2_src, W2_dst, att2_src, att2_dst, b2, Wl2, bl2)
    return out[:N]


# speedup vs baseline: 1.3925x; 1.0469x over previous
import functools

import jax
import jax.numpy as jnp
from jax import lax
from jax.experimental import pallas as pl
from jax.experimental.pallas import tpu as pltpu
from jax.experimental.pallas import tpu_sc as plsc

N = 10000
E = 320000
D = 128
NP = 10240
NW = 32
EPW = E // NW
CW = 128
CH = (EPW + CW - 1) // CW
EPP = CH * CW
DA = 144
NACC = 10112
ACC_PER_TILE = NACC // 16
PAD_DST = 10100
ROWS_PER_TILE = NP // 16


def _mm_body(x_ref, ws_ref, wl_ref, bl_ref, wd_ref, attd_ref, atts_ref,
             h_ref, skip_ref, as_ref, ad_ref, mas_ref, mad_ref):
    i = pl.program_id(0)
    xb = x_ref[...]
    h = jnp.dot(xb, ws_ref[...], preferred_element_type=jnp.float32)
    h_ref[:, :D] = h
    lane = jax.lax.broadcasted_iota(jnp.int32, (xb.shape[0], DA - D), 1)
    h_ref[:, D:] = jnp.where(lane == 0, 1.0, 0.0)
    skip_ref[...] = (jnp.dot(xb, wl_ref[...], preferred_element_type=jnp.float32)
                     + bl_ref[...][None, :])
    a_s = jnp.sum(h * atts_ref[...][None, :], axis=1)
    as_ref[...] = a_s
    wdv = jnp.sum(wd_ref[...] * attd_ref[...][None, :], axis=1)
    a_d = jnp.sum(xb * wdv[None, :], axis=1)
    ad_ref[...] = a_d

    @pl.when(i == 0)
    def _():
        mas_ref[0, 0] = -jnp.inf
        mad_ref[0, 0] = -jnp.inf

    mas_ref[0, 0] = jnp.maximum(mas_ref[0, 0], jnp.max(a_s))
    mad_ref[0, 0] = jnp.maximum(mad_ref[0, 0], jnp.max(a_d))


def _mm(x, w_src, wl, bl, w_dst, att_dst, att_src):
    blk = 512
    grid = NP // blk
    return pl.pallas_call(
        _mm_body,
        grid=(grid,),
        in_specs=[
            pl.BlockSpec((blk, D), lambda i: (i, 0)),
            pl.BlockSpec((D, D), lambda i: (0, 0)),
            pl.BlockSpec((D, D), lambda i: (0, 0)),
            pl.BlockSpec((D,), lambda i: (0,)),
            pl.BlockSpec((D, D), lambda i: (0, 0)),
            pl.BlockSpec((D,), lambda i: (0,)),
            pl.BlockSpec((D,), lambda i: (0,)),
        ],
        out_specs=[
            pl.BlockSpec((blk, DA), lambda i: (i, 0)),
            pl.BlockSpec((blk, D), lambda i: (i, 0)),
            pl.BlockSpec((blk,), lambda i: (i,)),
            pl.BlockSpec((blk,), lambda i: (i,)),
            pl.BlockSpec((1, 1), lambda i: (0, 0), memory_space=pltpu.SMEM),
            pl.BlockSpec((1, 1), lambda i: (0, 0), memory_space=pltpu.SMEM),
        ],
        out_shape=[
            jax.ShapeDtypeStruct((NP, DA), jnp.float32),
            jax.ShapeDtypeStruct((NP, D), jnp.float32),
            jax.ShapeDtypeStruct((NP,), jnp.float32),
            jax.ShapeDtypeStruct((NP,), jnp.float32),
            jax.ShapeDtypeStruct((1, 1), jnp.float32),
            jax.ShapeDtypeStruct((1, 1), jnp.float32),
        ],
    )(x, w_src, wl, bl, w_dst, att_dst, att_src)


NI = 3


def _sc_edge_body(h_hbm, as_hbm, ad_hbm, idx_hbm, c_hbm, zr_hbm,
                  acc_out,
                  acc_sh, asv, adv, idxv, pbuf, rows, cv, gsem, isem):
    ci = lax.axis_index("c")
    si = lax.axis_index("s")
    wid = ci * 16 + si

    pltpu.sync_copy(zr_hbm, acc_sh.at[pl.ds(si * ACC_PER_TILE, ACC_PER_TILE)])
    pltpu.sync_copy(as_hbm, asv)
    pltpu.sync_copy(ad_hbm, adv)
    pltpu.sync_copy(c_hbm, cv)
    plsc.subcore_barrier()

    cvec = cv[...]

    def compute_p(slot, pb):
        def vreg_body(k, c2):
            sidx = idxv[slot, 0, pl.ds(k * 16, 16)]
            didx = idxv[slot, 1, pl.ds(k * 16, 16)]
            e = plsc.load_gather(asv, [sidx]) + plsc.load_gather(adv, [didx])
            e = jnp.where(e > 0, e, 0.2 * e)
            pbuf[pb, pl.ds(k * 16, 16)] = jnp.exp(e - cvec)
            return c2

        lax.fori_loop(0, CW // 16, vreg_body, 0, unroll=True)

    def scale_rows(pb, half):
        @plsc.parallel_loop(half * (CW // 2), (half + 1) * (CW // 2), step=16)
        def row_body(g):
            pv = pbuf[pb, pl.ds(g, 16)]
            for l in range(16):
                ps = pv[l]
                for k2 in range(DA // 16):
                    rows[g + l, pl.ds(k2 * 16, 16)] = (
                        rows[g + l, pl.ds(k2 * 16, 16)] * ps)

    def gather_scale(slot, pb):
        HC = CW // 2
        rcp1 = pltpu.async_copy(h_hbm.at[idxv.at[slot, 0, pl.ds(0, HC)]],
                                rows.at[pl.ds(0, HC)], gsem)
        rcp2 = pltpu.async_copy(h_hbm.at[idxv.at[slot, 0, pl.ds(HC, HC)]],
                                rows.at[pl.ds(HC, HC)], gsem)
        rcp1.wait()
        scale_rows(pb, 0)
        rcp2.wait()
        scale_rows(pb, 1)

    pltpu.async_copy(idx_hbm.at[wid, 0], idxv.at[0], isem.at[0])
    pltpu.make_async_copy(idx_hbm.at[wid, 0], idxv.at[0], isem.at[0]).wait()
    pltpu.async_copy(idx_hbm.at[wid, 1], idxv.at[1], isem.at[1])
    compute_p(0, 0)
    gather_scale(0, 0)

    def chunk_body(i, carry):
        par = lax.rem(i, 2)
        nxt = lax.rem(i + 1, 2)
        slot = lax.rem(i, NI)
        nslot = lax.rem(i + 1, NI)
        fslot = lax.rem(i + 2, NI)

        scp = pltpu.async_copy(rows, acc_sh.at[idxv.at[slot, 1]], gsem,
                               add=True)

        @pl.when(i < CH - 1)
        def _():
            pltpu.make_async_copy(idx_hbm.at[wid, i + 1], idxv.at[nslot],
                                  isem.at[nslot]).wait()
            compute_p(nslot, nxt)

        scp.wait()

        @pl.when(i < CH - 1)
        def _():
            @pl.when(i < CH - 2)
            def _():
                pltpu.async_copy(idx_hbm.at[wid, i + 2], idxv.at[fslot],
                                 isem.at[fslot])

            gather_scale(nslot, nxt)

        return carry

    lax.fori_loop(0, CH, chunk_body, 0)
    plsc.subcore_barrier()

    sl = pl.ds(si * ACC_PER_TILE, ACC_PER_TILE)
    pltpu.sync_copy(acc_sh.at[sl], acc_out.at[ci, sl])


@functools.partial(
    pl.kernel,
    out_type=jax.ShapeDtypeStruct((2, NACC, DA), jnp.float32),
    mesh=plsc.VectorSubcoreMesh(core_axis_name="c", subcore_axis_name="s"),
    compiler_params=pltpu.CompilerParams(needs_layout_passes=False,
                                         use_tc_tiling_on_sc=False),
    scratch_types=[
        pltpu.VMEM_SHARED((NACC, DA), jnp.float32),
        pltpu.VMEM((NP,), jnp.float32),
        pltpu.VMEM((NP,), jnp.float32),
        pltpu.VMEM((NI, 2, CW), jnp.int32),
        pltpu.VMEM((2, CW), jnp.float32),
        pltpu.VMEM((CW, DA), jnp.float32),
        pltpu.VMEM((16,), jnp.float32),
        pltpu.SemaphoreType.DMA,
        pltpu.SemaphoreType.DMA((NI,)),
    ],
)
def _sc_edge(*refs):
    _sc_edge_body(*refs)


def _comb_body(acc_ref, skip_ref, b_ref, out_ref):
    full = acc_ref[0, :, :] + acc_ref[1, :, :]
    num = full[:, :D]
    dn = full[:, D:D + 1] + 1e-16
    h = num / dn + skip_ref[...] + b_ref[...][None, :]
    out_ref[...] = jnp.maximum(h, 0.0)


def _comb(acc, skip, b):
    blk = 1264
    return pl.pallas_call(
        _comb_body,
        grid=(NACC // blk,),
        in_specs=[
            pl.BlockSpec((2, blk, DA), lambda i: (0, i, 0)),
            pl.BlockSpec((blk, D), lambda i: (i, 0)),
            pl.BlockSpec((D,), lambda i: (0,)),
        ],
        out_specs=pl.BlockSpec((blk, D), lambda i: (i, 0)),
        out_shape=jax.ShapeDtypeStruct((NACC, D), jnp.float32),
    )(acc, skip, b)


def _layer(x_pad, idx3, zr, w_src, w_dst, att_src, att_dst, b, wl, bl):
    h, skip, a_s, a_d, mas, mad = _mm(x_pad, w_src, wl, bl, w_dst, att_dst, att_src)
    cb = mas[0, 0] + mad[0, 0]
    c = jnp.where(cb > 0, cb, 0.2 * cb)
    cvec = jnp.full((16,), c, jnp.float32)
    acc = _sc_edge(h, a_s, a_d, idx3, cvec, zr)
    out = _comb(acc, skip[:NACC], b)
    return jnp.pad(out, ((0, NP - NACC), (0, 0)))


def kernel(x, edge_index, W1_src, W1_dst, att1_src, att1_dst, b1, Wl1, bl1,
           W2_src, W2_dst, att2_src, att2_dst, b2, Wl2, bl2):
    x_pad = jnp.pad(x, ((0, NP - N), (0, 0)))
    src = edge_index[0].astype(jnp.int32).reshape(NW, EPW)
    dst = edge_index[1].astype(jnp.int32).reshape(NW, EPW)
    pad = ((0, 0), (0, EPP - EPW))
    src3 = jnp.pad(src, pad, constant_values=NP - 1).reshape(NW, CH, CW)
    dst3 = jnp.pad(dst, pad, constant_values=PAD_DST).reshape(NW, CH, CW)
    idx3 = jnp.stack([src3, dst3], axis=2)
    zr = jnp.zeros((ACC_PER_TILE, DA), jnp.float32)

    h = _layer(x_pad, idx3, zr,
               W1_src, W1_dst, att1_src, att1_dst, b1, Wl1, bl1)
    out = _layer(h, idx3, zr,
                 W2_src, W2_dst, att2_src, att2_dst, b2, Wl2, bl2)
    return out[:N]


# speedup vs baseline: 1.4444x; 1.0373x over previous
import functools

import jax
import jax.numpy as jnp
from jax import lax
from jax.experimental import pallas as pl
from jax.experimental.pallas import tpu as pltpu
from jax.experimental.pallas import tpu_sc as plsc

N = 10000
E = 320000
D = 128
NP = 10240
NW = 32
EPW = E // NW
CW = 128
CH = (EPW + CW - 1) // CW
EPP = CH * CW
DA = 144
NACC = 10112
ACC_PER_TILE = NACC // 16
PAD_DST = 10100
ROWS_PER_TILE = NP // 16


def _mm_body(x_ref, ws_ref, wl_ref, bl_ref, wd_ref, attd_ref, atts_ref,
             h_ref, skip_ref, as_ref, ad_ref, mas_ref, mad_ref):
    i = pl.program_id(0)
    xb = x_ref[...]
    h = jnp.dot(xb, ws_ref[...], preferred_element_type=jnp.float32)
    h_ref[:, :D] = h
    lane = jax.lax.broadcasted_iota(jnp.int32, (xb.shape[0], DA - D), 1)
    h_ref[:, D:] = jnp.where(lane == 0, 1.0, 0.0)
    skip_ref[...] = (jnp.dot(xb, wl_ref[...], preferred_element_type=jnp.float32)
                     + bl_ref[...][None, :])
    a_s = jnp.sum(h * atts_ref[...][None, :], axis=1)
    as_ref[...] = a_s
    wdv = jnp.sum(wd_ref[...] * attd_ref[...][None, :], axis=1)
    a_d = jnp.sum(xb * wdv[None, :], axis=1)
    ad_ref[...] = a_d

    @pl.when(i == 0)
    def _():
        mas_ref[0, 0] = -jnp.inf
        mad_ref[0, 0] = -jnp.inf

    mas_ref[0, 0] = jnp.maximum(mas_ref[0, 0], jnp.max(a_s))
    mad_ref[0, 0] = jnp.maximum(mad_ref[0, 0], jnp.max(a_d))


def _mm(x, w_src, wl, bl, w_dst, att_dst, att_src):
    blk = 512
    grid = NP // blk
    return pl.pallas_call(
        _mm_body,
        grid=(grid,),
        in_specs=[
            pl.BlockSpec((blk, D), lambda i: (i, 0)),
            pl.BlockSpec((D, D), lambda i: (0, 0)),
            pl.BlockSpec((D, D), lambda i: (0, 0)),
            pl.BlockSpec((D,), lambda i: (0,)),
            pl.BlockSpec((D, D), lambda i: (0, 0)),
            pl.BlockSpec((D,), lambda i: (0,)),
            pl.BlockSpec((D,), lambda i: (0,)),
        ],
        out_specs=[
            pl.BlockSpec((blk, DA), lambda i: (i, 0)),
            pl.BlockSpec((blk, D), lambda i: (i, 0)),
            pl.BlockSpec((blk,), lambda i: (i,)),
            pl.BlockSpec((blk,), lambda i: (i,)),
            pl.BlockSpec((1, 1), lambda i: (0, 0), memory_space=pltpu.SMEM),
            pl.BlockSpec((1, 1), lambda i: (0, 0), memory_space=pltpu.SMEM),
        ],
        out_shape=[
            jax.ShapeDtypeStruct((NP, DA), jnp.float32),
            jax.ShapeDtypeStruct((NP, D), jnp.float32),
            jax.ShapeDtypeStruct((NP,), jnp.float32),
            jax.ShapeDtypeStruct((NP,), jnp.float32),
            jax.ShapeDtypeStruct((1, 1), jnp.float32),
            jax.ShapeDtypeStruct((1, 1), jnp.float32),
        ],
    )(x, w_src, wl, bl, w_dst, att_dst, att_src)


NI = 3


def _sc_edge_body(h_hbm, as_hbm, ad_hbm, idx_hbm, c_hbm, zr_hbm,
                  acc_out,
                  acc_sh, asv, adv, idxv, pbuf, rows, cv, gsem, ssem, isem):
    ci = lax.axis_index("c")
    si = lax.axis_index("s")
    wid = ci * 16 + si

    pltpu.sync_copy(zr_hbm, acc_sh.at[pl.ds(si * ACC_PER_TILE, ACC_PER_TILE)])
    pltpu.sync_copy(as_hbm, asv)
    pltpu.sync_copy(ad_hbm, adv)
    pltpu.sync_copy(c_hbm, cv)
    plsc.subcore_barrier()

    cvec = cv[...]

    HC = CW // 2

    def compute_p(slot, pb):
        def vreg_body(k, c2):
            hf, ko = k // (HC // 16), k % (HC // 16)
            sidx = idxv[slot, 0, hf, pl.ds(ko * 16, 16)]
            didx = idxv[slot, 1, hf, pl.ds(ko * 16, 16)]
            e = plsc.load_gather(asv, [sidx]) + plsc.load_gather(adv, [didx])
            e = jnp.where(e > 0, e, 0.2 * e)
            pbuf[pb, pl.ds(k * 16, 16)] = jnp.exp(e - cvec)
            return c2

        lax.fori_loop(0, CW // 16, vreg_body, 0, unroll=True)

    def scale_rows(pb, half):
        @plsc.parallel_loop(half * (CW // 2), (half + 1) * (CW // 2), step=16)
        def row_body(g):
            pv = pbuf[pb, pl.ds(g, 16)]
            for l in range(16):
                ps = pv[l]
                for k2 in range(DA // 16):
                    rows[g + l, pl.ds(k2 * 16, 16)] = (
                        rows[g + l, pl.ds(k2 * 16, 16)] * ps)

    def gather_scale_scatter(slot, pb):
        rcp1 = pltpu.async_copy(h_hbm.at[idxv.at[slot, 0, 0]],
                                rows.at[pl.ds(0, HC)], gsem)
        rcp2 = pltpu.async_copy(h_hbm.at[idxv.at[slot, 0, 1]],
                                rows.at[pl.ds(HC, HC)], gsem)
        rcp1.wait()
        scale_rows(pb, 0)
        pltpu.async_copy(rows.at[pl.ds(0, HC)], acc_sh.at[idxv.at[slot, 1, 0]],
                         ssem, add=True)
        rcp2.wait()
        scale_rows(pb, 1)
        pltpu.async_copy(rows.at[pl.ds(HC, HC)], acc_sh.at[idxv.at[slot, 1, 1]],
                         ssem, add=True)

    def drain_scatters(slot):
        pltpu.make_async_copy(rows.at[pl.ds(0, HC)],
                              acc_sh.at[idxv.at[slot, 1, 0]], ssem).wait()
        pltpu.make_async_copy(rows.at[pl.ds(HC, HC)],
                              acc_sh.at[idxv.at[slot, 1, 1]], ssem).wait()

    pltpu.async_copy(idx_hbm.at[wid, 0], idxv.at[0], isem.at[0])
    pltpu.make_async_copy(idx_hbm.at[wid, 0], idxv.at[0], isem.at[0]).wait()
    pltpu.async_copy(idx_hbm.at[wid, 1], idxv.at[1], isem.at[1])
    compute_p(0, 0)
    gather_scale_scatter(0, 0)

    def chunk_body(i, carry):
        nxt = lax.rem(i + 1, 2)
        slot = lax.rem(i, NI)
        nslot = lax.rem(i + 1, NI)
        fslot = lax.rem(i + 2, NI)

        @pl.when(i < CH - 1)
        def _():
            pltpu.make_async_copy(idx_hbm.at[wid, i + 1], idxv.at[nslot],
                                  isem.at[nslot]).wait()
            compute_p(nslot, nxt)

        drain_scatters(slot)

        @pl.when(i < CH - 1)
        def _():
            @pl.when(i < CH - 2)
            def _():
                pltpu.async_copy(idx_hbm.at[wid, i + 2], idxv.at[fslot],
                                 isem.at[fslot])

            gather_scale_scatter(nslot, nxt)

        return carry

    lax.fori_loop(0, CH, chunk_body, 0)
    plsc.subcore_barrier()

    sl = pl.ds(si * ACC_PER_TILE, ACC_PER_TILE)
    pltpu.sync_copy(acc_sh.at[sl], acc_out.at[ci, sl])


@functools.partial(
    pl.kernel,
    out_type=jax.ShapeDtypeStruct((2, NACC, DA), jnp.float32),
    mesh=plsc.VectorSubcoreMesh(core_axis_name="c", subcore_axis_name="s"),
    compiler_params=pltpu.CompilerParams(needs_layout_passes=False,
                                         use_tc_tiling_on_sc=False),
    scratch_types=[
        pltpu.VMEM_SHARED((NACC, DA), jnp.float32),
        pltpu.VMEM((NP,), jnp.float32),
        pltpu.VMEM((NP,), jnp.float32),
        pltpu.VMEM((NI, 2, 2, CW // 2), jnp.int32),
        pltpu.VMEM((2, CW), jnp.float32),
        pltpu.VMEM((CW, DA), jnp.float32),
        pltpu.VMEM((16,), jnp.float32),
        pltpu.SemaphoreType.DMA,
        pltpu.SemaphoreType.DMA,
        pltpu.SemaphoreType.DMA((NI,)),
    ],
)
def _sc_edge(*refs):
    _sc_edge_body(*refs)


def _comb_body(acc_ref, skip_ref, b_ref, out_ref):
    full = acc_ref[0, :, :] + acc_ref[1, :, :]
    num = full[:, :D]
    dn = full[:, D:D + 1] + 1e-16
    h = num / dn + skip_ref[...] + b_ref[...][None, :]
    out_ref[...] = jnp.maximum(h, 0.0)


def _comb(acc, skip, b):
    blk = 1264
    return pl.pallas_call(
        _comb_body,
        grid=(NACC // blk,),
        in_specs=[
            pl.BlockSpec((2, blk, DA), lambda i: (0, i, 0)),
            pl.BlockSpec((blk, D), lambda i: (i, 0)),
            pl.BlockSpec((D,), lambda i: (0,)),
        ],
        out_specs=pl.BlockSpec((blk, D), lambda i: (i, 0)),
        out_shape=jax.ShapeDtypeStruct((NACC, D), jnp.float32),
    )(acc, skip, b)


def _layer(x_pad, idx3, zr, w_src, w_dst, att_src, att_dst, b, wl, bl):
    h, skip, a_s, a_d, mas, mad = _mm(x_pad, w_src, wl, bl, w_dst, att_dst, att_src)
    cb = mas[0, 0] + mad[0, 0]
    c = jnp.where(cb > 0, cb, 0.2 * cb)
    cvec = jnp.full((16,), c, jnp.float32)
    acc = _sc_edge(h, a_s, a_d, idx3, cvec, zr)
    out = _comb(acc, skip[:NACC], b)
    return jnp.pad(out, ((0, NP - NACC), (0, 0)))


def kernel(x, edge_index, W1_src, W1_dst, att1_src, att1_dst, b1, Wl1, bl1,
           W2_src, W2_dst, att2_src, att2_dst, b2, Wl2, bl2):
    x_pad = jnp.pad(x, ((0, NP - N), (0, 0)))
    src = edge_index[0].astype(jnp.int32).reshape(NW, EPW)
    dst = edge_index[1].astype(jnp.int32).reshape(NW, EPW)
    pad = ((0, 0), (0, EPP - EPW))
    src3 = jnp.pad(src, pad, constant_values=NP - 1).reshape(NW, CH, 2, CW // 2)
    dst3 = jnp.pad(dst, pad, constant_values=PAD_DST).reshape(NW, CH, 2, CW // 2)
    idx3 = jnp.stack([src3, dst3], axis=2)
    zr = jnp.zeros((ACC_PER_TILE, DA), jnp.float32)

    h = _layer(x_pad, idx3, zr,
               W1_src, W1_dst, att1_src, att1_dst, b1, Wl1, bl1)
    out = _layer(h, idx3, zr,
                 W2_src, W2_dst, att2_src, att2_dst, b2, Wl2, bl2)
    return out[:N]


# speedup vs baseline: 1.4675x; 1.0160x over previous
import functools

import jax
import jax.numpy as jnp
from jax import lax
from jax.experimental import pallas as pl
from jax.experimental.pallas import tpu as pltpu
from jax.experimental.pallas import tpu_sc as plsc

N = 10000
E = 320000
D = 128
NP = 10240
NW = 32
EPW = E // NW
CW = 128
CH = (EPW + CW - 1) // CW
EPP = CH * CW
DA = 144
NACC = 10112
ACC_PER_TILE = NACC // 16
PAD_DST = 10100
ROWS_PER_TILE = NP // 16


def _mm_body(x_ref, ws_ref, wl_ref, bl_ref, wd_ref, attd_ref, atts_ref,
             h_ref, skip_ref, as_ref, ad_ref, mas_ref, mad_ref):
    i = pl.program_id(0)
    xb = x_ref[...]
    h = jnp.dot(xb, ws_ref[...], preferred_element_type=jnp.float32)
    h_ref[:, :D] = h
    lane = jax.lax.broadcasted_iota(jnp.int32, (xb.shape[0], DA - D), 1)
    h_ref[:, D:] = jnp.where(lane == 0, 1.0, 0.0)
    skip_ref[...] = (jnp.dot(xb, wl_ref[...], preferred_element_type=jnp.float32)
                     + bl_ref[...][None, :])
    a_s = jnp.sum(h * atts_ref[...][None, :], axis=1)
    as_ref[...] = a_s
    wdv = jnp.sum(wd_ref[...] * attd_ref[...][None, :], axis=1)
    a_d = jnp.sum(xb * wdv[None, :], axis=1)
    ad_ref[...] = a_d

    @pl.when(i == 0)
    def _():
        mas_ref[0, 0] = -jnp.inf
        mad_ref[0, 0] = -jnp.inf

    mas_ref[0, 0] = jnp.maximum(mas_ref[0, 0], jnp.max(a_s))
    mad_ref[0, 0] = jnp.maximum(mad_ref[0, 0], jnp.max(a_d))


def _mm(x, w_src, wl, bl, w_dst, att_dst, att_src):
    blk = 512
    grid = NP // blk
    return pl.pallas_call(
        _mm_body,
        grid=(grid,),
        in_specs=[
            pl.BlockSpec((blk, D), lambda i: (i, 0)),
            pl.BlockSpec((D, D), lambda i: (0, 0)),
            pl.BlockSpec((D, D), lambda i: (0, 0)),
            pl.BlockSpec((D,), lambda i: (0,)),
            pl.BlockSpec((D, D), lambda i: (0, 0)),
            pl.BlockSpec((D,), lambda i: (0,)),
            pl.BlockSpec((D,), lambda i: (0,)),
        ],
        out_specs=[
            pl.BlockSpec((blk, DA), lambda i: (i, 0)),
            pl.BlockSpec((blk, D), lambda i: (i, 0)),
            pl.BlockSpec((blk,), lambda i: (i,)),
            pl.BlockSpec((blk,), lambda i: (i,)),
            pl.BlockSpec((1, 1), lambda i: (0, 0), memory_space=pltpu.SMEM),
            pl.BlockSpec((1, 1), lambda i: (0, 0), memory_space=pltpu.SMEM),
        ],
        out_shape=[
            jax.ShapeDtypeStruct((NP, DA), jnp.float32),
            jax.ShapeDtypeStruct((NP, D), jnp.float32),
            jax.ShapeDtypeStruct((NP,), jnp.float32),
            jax.ShapeDtypeStruct((NP,), jnp.float32),
            jax.ShapeDtypeStruct((1, 1), jnp.float32),
            jax.ShapeDtypeStruct((1, 1), jnp.float32),
        ],
    )(x, w_src, wl, bl, w_dst, att_dst, att_src)


NI = 3
NQ = 4


def _sc_edge_body(h_hbm, as_hbm, ad_hbm, idx_hbm, c_hbm, zr_hbm,
                  acc_out,
                  acc_sh, asv, adv, idxv, pbuf, rows, cv, gsem, ssem, isem):
    ci = lax.axis_index("c")
    si = lax.axis_index("s")
    wid = ci * 16 + si

    pltpu.sync_copy(zr_hbm, acc_sh.at[pl.ds(si * ACC_PER_TILE, ACC_PER_TILE)])
    pltpu.sync_copy(as_hbm, asv)
    pltpu.sync_copy(ad_hbm, adv)
    pltpu.sync_copy(c_hbm, cv)
    plsc.subcore_barrier()

    cvec = cv[...]

    QC = CW // NQ

    def compute_p(slot, pb):
        def vreg_body(k, c2):
            q, ko = k // (QC // 16), k % (QC // 16)
            sidx = idxv[slot, 0, q, pl.ds(ko * 16, 16)]
            didx = idxv[slot, 1, q, pl.ds(ko * 16, 16)]
            e = plsc.load_gather(asv, [sidx]) + plsc.load_gather(adv, [didx])
            e = jnp.where(e > 0, e, 0.2 * e)
            pbuf[pb, pl.ds(k * 16, 16)] = jnp.exp(e - cvec)
            return c2

        lax.fori_loop(0, CW // 16, vreg_body, 0, unroll=True)

    def scale_rows(pb, q):
        @plsc.parallel_loop(q * QC, (q + 1) * QC, step=16)
        def row_body(g):
            pv = pbuf[pb, pl.ds(g, 16)]
            for l in range(16):
                ps = pv[l]
                for k2 in range(DA // 16):
                    rows[g + l, pl.ds(k2 * 16, 16)] = (
                        rows[g + l, pl.ds(k2 * 16, 16)] * ps)

    def gather_scale_scatter(slot, pb):
        rcps = [pltpu.async_copy(h_hbm.at[idxv.at[slot, 0, q]],
                                 rows.at[pl.ds(q * QC, QC)], gsem)
                for q in range(NQ)]
        for q in range(NQ):
            rcps[q].wait()
            scale_rows(pb, q)
            pltpu.async_copy(rows.at[pl.ds(q * QC, QC)],
                             acc_sh.at[idxv.at[slot, 1, q]], ssem, add=True)

    def drain_scatters(slot):
        for q in range(NQ):
            pltpu.make_async_copy(rows.at[pl.ds(q * QC, QC)],
                                  acc_sh.at[idxv.at[slot, 1, q]], ssem).wait()

    pltpu.async_copy(idx_hbm.at[wid, 0], idxv.at[0], isem.at[0])
    pltpu.make_async_copy(idx_hbm.at[wid, 0], idxv.at[0], isem.at[0]).wait()
    pltpu.async_copy(idx_hbm.at[wid, 1], idxv.at[1], isem.at[1])
    compute_p(0, 0)
    gather_scale_scatter(0, 0)

    def chunk_body(i, carry):
        nxt = lax.rem(i + 1, 2)
        slot = lax.rem(i, NI)
        nslot = lax.rem(i + 1, NI)
        fslot = lax.rem(i + 2, NI)

        @pl.when(i < CH - 1)
        def _():
            pltpu.make_async_copy(idx_hbm.at[wid, i + 1], idxv.at[nslot],
                                  isem.at[nslot]).wait()
            compute_p(nslot, nxt)

        drain_scatters(slot)

        @pl.when(i < CH - 1)
        def _():
            @pl.when(i < CH - 2)
            def _():
                pltpu.async_copy(idx_hbm.at[wid, i + 2], idxv.at[fslot],
                                 isem.at[fslot])

            gather_scale_scatter(nslot, nxt)

        return carry

    lax.fori_loop(0, CH, chunk_body, 0)
    plsc.subcore_barrier()

    sl = pl.ds(si * ACC_PER_TILE, ACC_PER_TILE)
    pltpu.sync_copy(acc_sh.at[sl], acc_out.at[ci, sl])


@functools.partial(
    pl.kernel,
    out_type=jax.ShapeDtypeStruct((2, NACC, DA), jnp.float32),
    mesh=plsc.VectorSubcoreMesh(core_axis_name="c", subcore_axis_name="s"),
    compiler_params=pltpu.CompilerParams(needs_layout_passes=False,
                                         use_tc_tiling_on_sc=False),
    scratch_types=[
        pltpu.VMEM_SHARED((NACC, DA), jnp.float32),
        pltpu.VMEM((NP,), jnp.float32),
        pltpu.VMEM((NP,), jnp.float32),
        pltpu.VMEM((NI, 2, NQ, CW // NQ), jnp.int32),
        pltpu.VMEM((2, CW), jnp.float32),
        pltpu.VMEM((CW, DA), jnp.float32),
        pltpu.VMEM((16,), jnp.float32),
        pltpu.SemaphoreType.DMA,
        pltpu.SemaphoreType.DMA,
        pltpu.SemaphoreType.DMA((NI,)),
    ],
)
def _sc_edge(*refs):
    _sc_edge_body(*refs)


def _comb_body(acc_ref, skip_ref, b_ref, out_ref):
    full = acc_ref[0, :, :] + acc_ref[1, :, :]
    num = full[:, :D]
    dn = full[:, D:D + 1] + 1e-16
    h = num / dn + skip_ref[...] + b_ref[...][None, :]
    out_ref[...] = jnp.maximum(h, 0.0)


def _comb(acc, skip, b):
    blk = 1264
    return pl.pallas_call(
        _comb_body,
        grid=(NACC // blk,),
        in_specs=[
            pl.BlockSpec((2, blk, DA), lambda i: (0, i, 0)),
            pl.BlockSpec((blk, D), lambda i: (i, 0)),
            pl.BlockSpec((D,), lambda i: (0,)),
        ],
        out_specs=pl.BlockSpec((blk, D), lambda i: (i, 0)),
        out_shape=jax.ShapeDtypeStruct((NACC, D), jnp.float32),
    )(acc, skip, b)


def _layer(x_pad, idx3, zr, w_src, w_dst, att_src, att_dst, b, wl, bl):
    h, skip, a_s, a_d, mas, mad = _mm(x_pad, w_src, wl, bl, w_dst, att_dst, att_src)
    cb = mas[0, 0] + mad[0, 0]
    c = jnp.where(cb > 0, cb, 0.2 * cb)
    cvec = jnp.full((16,), c, jnp.float32)
    acc = _sc_edge(h, a_s, a_d, idx3, cvec, zr)
    out = _comb(acc, skip[:NACC], b)
    return jnp.pad(out, ((0, NP - NACC), (0, 0)))


def kernel(x, edge_index, W1_src, W1_dst, att1_src, att1_dst, b1, Wl1, bl1,
           W2_src, W2_dst, att2_src, att2_dst, b2, Wl2, bl2):
    x_pad = jnp.pad(x, ((0, NP - N), (0, 0)))
    src = edge_index[0].astype(jnp.int32).reshape(NW, EPW)
    dst = edge_index[1].astype(jnp.int32).reshape(NW, EPW)
    pad = ((0, 0), (0, EPP - EPW))
    src3 = jnp.pad(src, pad, constant_values=NP - 1).reshape(NW, CH, NQ, CW // NQ)
    dst3 = jnp.pad(dst, pad, constant_values=PAD_DST).reshape(NW, CH, NQ, CW // NQ)
    idx3 = jnp.stack([src3, dst3], axis=2)
    zr = jnp.zeros((ACC_PER_TILE, DA), jnp.float32)

    h = _layer(x_pad, idx3, zr,
               W1_src, W1_dst, att1_src, att1_dst, b1, Wl1, bl1)
    out = _layer(h, idx3, zr,
                 W2_src, W2_dst, att2_src, att2_dst, b2, Wl2, bl2)
    return out[:N]


# speedup vs baseline: 1.4683x; 1.0005x over previous
import functools

import jax
import jax.numpy as jnp
from jax import lax
from jax.experimental import pallas as pl
from jax.experimental.pallas import tpu as pltpu
from jax.experimental.pallas import tpu_sc as plsc

N = 10000
E = 320000
D = 128
NP = 10240
NW = 32
EPW = E // NW
CW = 128
CH = (EPW + CW - 1) // CW
EPP = CH * CW
DA = 144
NACC = 10112
ACC_PER_TILE = NACC // 16
PAD_DST = 10100


def _mm_body(x_ref, ws_ref, wl_ref, bl_ref, wd_ref, attd_ref, atts_ref,
             h_ref, skip_ref, as_ref, ad_ref, mas_ref, mad_ref):
    i = pl.program_id(0)
    xb = x_ref[...]
    h = jnp.dot(xb, ws_ref[...], preferred_element_type=jnp.float32)
    h_ref[:, :D] = h
    lane = jax.lax.broadcasted_iota(jnp.int32, (xb.shape[0], DA - D), 1)
    h_ref[:, D:] = jnp.where(lane == 0, 1.0, 0.0)
    skip_ref[...] = (jnp.dot(xb, wl_ref[...], preferred_element_type=jnp.float32)
                     + bl_ref[...][None, :])
    a_s = jnp.sum(h * atts_ref[...][None, :], axis=1)
    as_ref[...] = a_s
    wdv = jnp.sum(wd_ref[...] * attd_ref[...][None, :], axis=1)
    a_d = jnp.sum(xb * wdv[None, :], axis=1)
    ad_ref[...] = a_d

    @pl.when(i == 0)
    def _():
        mas_ref[0, 0] = -jnp.inf
        mad_ref[0, 0] = -jnp.inf

    mas_ref[0, 0] = jnp.maximum(mas_ref[0, 0], jnp.max(a_s))
    mad_ref[0, 0] = jnp.maximum(mad_ref[0, 0], jnp.max(a_d))


def _mm(x, w_src, wl, bl, w_dst, att_dst, att_src):
    blk = 512
    grid = NP // blk
    return pl.pallas_call(
        _mm_body,
        grid=(grid,),
        in_specs=[
            pl.BlockSpec((blk, D), lambda i: (i, 0)),
            pl.BlockSpec((D, D), lambda i: (0, 0)),
            pl.BlockSpec((D, D), lambda i: (0, 0)),
            pl.BlockSpec((D,), lambda i: (0,)),
            pl.BlockSpec((D, D), lambda i: (0, 0)),
            pl.BlockSpec((D,), lambda i: (0,)),
            pl.BlockSpec((D,), lambda i: (0,)),
        ],
        out_specs=[
            pl.BlockSpec((blk, DA), lambda i: (i, 0)),
            pl.BlockSpec((blk, D), lambda i: (i, 0)),
            pl.BlockSpec((blk,), lambda i: (i,)),
            pl.BlockSpec((blk,), lambda i: (i,)),
            pl.BlockSpec((1, 1), lambda i: (0, 0), memory_space=pltpu.SMEM),
            pl.BlockSpec((1, 1), lambda i: (0, 0), memory_space=pltpu.SMEM),
        ],
        out_shape=[
            jax.ShapeDtypeStruct((NP, DA), jnp.float32),
            jax.ShapeDtypeStruct((NP, D), jnp.float32),
            jax.ShapeDtypeStruct((NP,), jnp.float32),
            jax.ShapeDtypeStruct((NP,), jnp.float32),
            jax.ShapeDtypeStruct((1, 1), jnp.float32),
            jax.ShapeDtypeStruct((1, 1), jnp.float32),
        ],
    )(x, w_src, wl, bl, w_dst, att_dst, att_src)


NI = 3
NQ = 4


def _sc_edge_body(h_hbm, as_hbm, ad_hbm, idx_hbm, c_hbm, zr_hbm,
                  acc_out,
                  acc_sh, asv, adv, idxv, pbuf, rows, cv, gsem, ssem, isem):
    ci = lax.axis_index("c")
    si = lax.axis_index("s")
    wid = ci * 16 + si

    pltpu.sync_copy(zr_hbm, acc_sh.at[pl.ds(si * ACC_PER_TILE, ACC_PER_TILE)])
    pltpu.sync_copy(as_hbm, asv)
    pltpu.sync_copy(ad_hbm, adv)
    pltpu.sync_copy(c_hbm, cv)
    plsc.subcore_barrier()

    cvec = cv[...]

    QC = CW // NQ

    def compute_p(slot, pb):
        def vreg_body(k, c2):
            q, ko = k // (QC // 16), k % (QC // 16)
            sidx = idxv[slot, 0, q, pl.ds(ko * 16, 16)]
            didx = idxv[slot, 1, q, pl.ds(ko * 16, 16)]
            e = plsc.load_gather(asv, [sidx]) + plsc.load_gather(adv, [didx])
            e = jnp.where(e > 0, e, 0.2 * e)
            pbuf[pb, pl.ds(k * 16, 16)] = jnp.exp(e - cvec)
            return c2

        lax.fori_loop(0, CW // 16, vreg_body, 0, unroll=True)

    def scale_rows(pb, q):
        @plsc.parallel_loop(q * QC, (q + 1) * QC, step=16)
        def row_body(g):
            pv = pbuf[pb, pl.ds(g, 16)]
            for l in range(16):
                ps = pv[l]
                for k2 in range(DA // 16):
                    rows[g + l, pl.ds(k2 * 16, 16)] = (
                        rows[g + l, pl.ds(k2 * 16, 16)] * ps)

    def gather_scale_scatter(slot, pb):
        rcps = [pltpu.async_copy(h_hbm.at[idxv.at[slot, 0, q]],
                                 rows.at[pl.ds(q * QC, QC)], gsem)
                for q in range(NQ)]
        for q in range(NQ):
            rcps[q].wait()
            scale_rows(pb, q)
            pltpu.async_copy(rows.at[pl.ds(q * QC, QC)],
                             acc_sh.at[idxv.at[slot, 1, q]], ssem, add=True)

    def drain_scatters(slot):
        for q in range(NQ):
            pltpu.make_async_copy(rows.at[pl.ds(q * QC, QC)],
                                  acc_sh.at[idxv.at[slot, 1, q]], ssem).wait()

    pltpu.async_copy(idx_hbm.at[wid, 0], idxv.at[0], isem.at[0])
    pltpu.make_async_copy(idx_hbm.at[wid, 0], idxv.at[0], isem.at[0]).wait()
    pltpu.async_copy(idx_hbm.at[wid, 1], idxv.at[1], isem.at[1])
    compute_p(0, 0)
    gather_scale_scatter(0, 0)

    def chunk_body(i, carry):
        nxt = lax.rem(i + 1, 2)
        slot = lax.rem(i, NI)
        nslot = lax.rem(i + 1, NI)
        fslot = lax.rem(i + 2, NI)

        @pl.when(i < CH - 1)
        def _():
            pltpu.make_async_copy(idx_hbm.at[wid, i + 1], idxv.at[nslot],
                                  isem.at[nslot]).wait()
            compute_p(nslot, nxt)

        drain_scatters(slot)

        @pl.when(i < CH - 1)
        def _():
            @pl.when(i < CH - 2)
            def _():
                pltpu.async_copy(idx_hbm.at[wid, i + 2], idxv.at[fslot],
                                 isem.at[fslot])

            gather_scale_scatter(nslot, nxt)

        return carry

    lax.fori_loop(0, CH, chunk_body, 0)
    plsc.subcore_barrier()

    sl = pl.ds(si * ACC_PER_TILE, ACC_PER_TILE)
    pltpu.sync_copy(acc_sh.at[sl], acc_out.at[ci, sl])


@functools.partial(
    pl.kernel,
    out_type=jax.ShapeDtypeStruct((2, NACC, DA), jnp.float32),
    mesh=plsc.VectorSubcoreMesh(core_axis_name="c", subcore_axis_name="s"),
    compiler_params=pltpu.CompilerParams(needs_layout_passes=False,
                                         use_tc_tiling_on_sc=False),
    scratch_types=[
        pltpu.VMEM_SHARED((NACC, DA), jnp.float32),
        pltpu.VMEM((NP,), jnp.float32),
        pltpu.VMEM((NP,), jnp.float32),
        pltpu.VMEM((NI, 2, NQ, CW // NQ), jnp.int32),
        pltpu.VMEM((2, CW), jnp.float32),
        pltpu.VMEM((CW, DA), jnp.float32),
        pltpu.VMEM((16,), jnp.float32),
        pltpu.SemaphoreType.DMA,
        pltpu.SemaphoreType.DMA,
        pltpu.SemaphoreType.DMA((NI,)),
    ],
)
def _sc_edge(*refs):
    _sc_edge_body(*refs)


def _comb_body(acc_ref, skip_ref, b_ref, out_ref):
    full = acc_ref[0, :, :] + acc_ref[1, :, :]
    num = full[:, :D]
    dn = full[:, D:D + 1] + 1e-16
    h = num / dn + skip_ref[...] + b_ref[...][None, :]
    out_ref[...] = jnp.maximum(h, 0.0)


def _comb(acc, skip, b):
    blk = 1264
    return pl.pallas_call(
        _comb_body,
        grid=(NACC // blk,),
        in_specs=[
            pl.BlockSpec((2, blk, DA), lambda i: (0, i, 0)),
            pl.BlockSpec((blk, D), lambda i: (i, 0)),
            pl.BlockSpec((D,), lambda i: (0,)),
        ],
        out_specs=pl.BlockSpec((blk, D), lambda i: (i, 0)),
        out_shape=jax.ShapeDtypeStruct((NACC, D), jnp.float32),
    )(acc, skip, b)


def _layer(x_pad, idx3, zr, w_src, w_dst, att_src, att_dst, b, wl, bl):
    h, skip, a_s, a_d, mas, mad = _mm(x_pad, w_src, wl, bl, w_dst, att_dst, att_src)
    cb = mas[0, 0] + mad[0, 0]
    c = jnp.where(cb > 0, cb, 0.2 * cb)
    cvec = jnp.full((16,), c, jnp.float32)
    acc = _sc_edge(h, a_s, a_d, idx3, cvec, zr)
    out = _comb(acc, skip[:NACC], b)
    return jnp.pad(out, ((0, NP - NACC), (0, 0)))


def kernel(x, edge_index, W1_src, W1_dst, att1_src, att1_dst, b1, Wl1, bl1,
           W2_src, W2_dst, att2_src, att2_dst, b2, Wl2, bl2):
    x_pad = jnp.pad(x, ((0, NP - N), (0, 0)))
    src = edge_index[0].astype(jnp.int32).reshape(NW, EPW)
    dst = edge_index[1].astype(jnp.int32).reshape(NW, EPW)
    pad = ((0, 0), (0, EPP - EPW))
    src3 = jnp.pad(src, pad, constant_values=NP - 1).reshape(NW, CH, NQ, CW // NQ)
    dst3 = jnp.pad(dst, pad, constant_values=PAD_DST).reshape(NW, CH, NQ, CW // NQ)
    idx3 = jnp.stack([src3, dst3], axis=2)
    zr = jnp.zeros((ACC_PER_TILE, DA), jnp.float32)

    h = _layer(x_pad, idx3, zr,
               W1_src, W1_dst, att1_src, att1_dst, b1, Wl1, bl1)
    out = _layer(h, idx3, zr,
                 W2_src, W2_dst, att2_src, att2_dst, b2, Wl2, bl2)
    return out[:N]
